# Initial kernel scaffold; baseline (speedup 1.0000x reference)
import jax
import jax.numpy as jnp
from jax.experimental import pallas as pl


def kernel(x, edge_index, batch, Ws, a_src, a_dst, W1, b1, W2, b2, W3, b3):
    raise NotImplementedError("write your pallas kernel here")


# speedup vs baseline: 1.0010x; 1.0010x over previous
import functools

import jax
import jax.numpy as jnp
import numpy as np
from jax.experimental import pallas as pl
from jax.experimental.pallas import tpu as pltpu

N = 10000
E = 320000
D = 128
H = 4
DH = D // H
L = 3
G = 64


def _layer_dense_body(hin_ref, w_ref, asrc_ref, adst_ref, h_ref, es_ref, ed_ref):
    h = jnp.dot(hin_ref[...], w_ref[...], preferred_element_type=jnp.float32)
    h_ref[...] = h
    es_ref[...] = jnp.dot(h, asrc_ref[...], preferred_element_type=jnp.float32)
    ed_ref[...] = jnp.dot(h, adst_ref[...], preferred_element_type=jnp.float32)


def _layer_dense(hin, W, Asrc, Adst):
    return pl.pallas_call(
        _layer_dense_body,
        out_shape=(
            jax.ShapeDtypeStruct((N, D), jnp.float32),
            jax.ShapeDtypeStruct((N, H), jnp.float32),
            jax.ShapeDtypeStruct((N, H), jnp.float32),
        ),
    )(hin, W, Asrc, Adst)


def _mlp_body(e_ref, w1_ref, b1_ref, w2_ref, b2_ref, w3_ref, b3_ref, q_ref):
    h1 = jnp.maximum(
        jnp.dot(e_ref[...], w1_ref[...], preferred_element_type=jnp.float32)
        + b1_ref[...], 0.0)
    h2 = jnp.maximum(
        jnp.dot(h1, w2_ref[...], preferred_element_type=jnp.float32)
        + b2_ref[...], 0.0)
    q_ref[...] = (
        jnp.dot(h2, w3_ref[...], preferred_element_type=jnp.float32)
        + b3_ref[...])


def _mlp(e, W1, b1, W2, b2, W3, b3):
    out = pl.pallas_call(
        _mlp_body,
        out_shape=jax.ShapeDtypeStruct((G, 1), jnp.float32),
    )(e, W1, b1[None, :], W2, b2[None, :], W3, b3[None, :])
    return out.reshape(-1)


def kernel(x, edge_index, batch, Ws, a_src, a_dst, W1, b1, W2, b2, W3, b3):
    src = edge_index[0]
    dst = edge_index[1]
    counts = jax.ops.segment_sum(jnp.ones((N,), dtype=jnp.float32), batch,
                                 num_segments=G)
    head = jnp.arange(D, dtype=jnp.int32) // DH
    blockmask = (head[:, None] == jnp.arange(H, dtype=jnp.int32)[None, :])
    h_in = x
    readouts = []
    for l in range(L):
        Asrc = jnp.where(blockmask, a_src[l].reshape(D)[:, None], 0.0)
        Adst = jnp.where(blockmask, a_dst[l].reshape(D)[:, None], 0.0)
        h, e_s, e_d = _layer_dense(h_in, Ws[l], Asrc, Adst)
        hh = h.reshape(N, H, DH)
        logits = jax.nn.leaky_relu(e_s[src] + e_d[dst], 0.2)
        m = jax.ops.segment_max(logits, dst, num_segments=N)
        m = jnp.where(jnp.isfinite(m), m, 0.0)
        ea = jnp.exp(logits - m[dst])
        denom = jax.ops.segment_sum(ea, dst, num_segments=N)
        alpha = ea / (denom[dst] + 1e-16)
        msg = hh[src] * alpha[:, :, None]
        agg = jax.ops.segment_sum(msg, dst, num_segments=N).reshape(N, D)
        h_in = jax.nn.elu(agg)
        ssum = jax.ops.segment_sum(h_in, batch, num_segments=G)
        meanp = ssum / jnp.maximum(counts, 1.0)[:, None]
        maxp = jax.ops.segment_max(h_in, batch, num_segments=G)
        maxp = jnp.where(counts[:, None] > 0, maxp, 0.0)
        readouts.append(meanp)
        readouts.append(maxp)
    e = jnp.concatenate(readouts, axis=1)
    h1 = jax.nn.relu(e @ W1 + b1)
    h2 = jax.nn.relu(h1 @ W2 + b2)
    return (h2 @ W3 + b3).reshape(-1)
